# baseline (speedup 1.0000x reference)
import functools

import jax
import jax.numpy as jnp
from jax import lax
from jax.experimental import pallas as pl
from jax.experimental.pallas import tpu as pltpu
from jax.experimental.pallas import tpu_sc as plsc

N_SEG = 64
LANES = 16
NC, NS = 2, 16
NW = NC * NS
ROWS = 16384
COLS = 2048
R_CHUNK = 16
N_CHUNK = ROWS // R_CHUNK // NW
CPG = 1024 // R_CHUNK
W_TILE = 512
NT = COLS // W_TILE


def _round_idx(num):
    q = num >> 6
    rem = num & 63
    tie_up = (rem == 32) & ((q & 1) == 0)
    inc = jnp.where((rem > 32) | tie_up, 1, 0)
    return q + inc


def _make_sc_kernel():
    mesh = plsc.VectorSubcoreMesh(core_axis_name="c", subcore_axis_name="s")

    @functools.partial(
        pl.kernel,
        mesh=mesh,
        compiler_params=pltpu.CompilerParams(needs_layout_passes=False),
        out_type=jax.ShapeDtypeStruct((ROWS, N_SEG), jnp.float32),
        scratch_types=[
            pltpu.VMEM((LANES,), jnp.int32),
            pltpu.VMEM((R_CHUNK, COLS), jnp.float32),
            pltpu.VMEM((R_CHUNK, COLS), jnp.float32),
            pltpu.VMEM((R_CHUNK, COLS), jnp.float32),
            pltpu.VMEM((R_CHUNK, N_SEG), jnp.float32),
            pltpu.VMEM((R_CHUNK, N_SEG), jnp.float32),
            pltpu.VMEM((R_CHUNK, N_SEG), jnp.float32),
            pltpu.SMEM((LANES,), jnp.int32),
            pltpu.SMEM((LANES,), jnp.int32),
            pltpu.SemaphoreType.DMA,
            pltpu.SemaphoreType.DMA,
            pltpu.SemaphoreType.DMA,
            pltpu.SemaphoreType.DMA,
            pltpu.SemaphoreType.DMA,
            pltpu.SemaphoreType.DMA,
        ],
    )
    def k(inp_hbm, len_hbm, out_hbm, len_v,
          in0, in1, in2, ob0, ob1, ob2, t_tab, nt_tab,
          is0, is1, is2, os0, os1, os2):
        wid = lax.axis_index("s") * NC + lax.axis_index("c")

        pltpu.sync_copy(len_hbm, len_v)
        lane = lax.iota(jnp.int32, LANES)

        ins = [in0, in1, in2]
        obs = [ob0, ob1, ob2]
        isems = [is0, is1, is2]
        osems = [os0, os1, os2]

        def fill_tab(g, carry):
            T0 = jnp.max(jnp.where(lane == g, len_v[...], 0))
            T = jnp.where(T0 < 2 * N_SEG, (2 * N_SEG // T0 + 1) * T0, T0)
            idx_max = _round_idx((T - 1) * (N_SEG - 1))
            t_tab[g] = T
            nt_tab[g] = (idx_max >> 9) + 1
            return carry
        lax.fori_loop(0, LANES, fill_tab, 0)

        def chunk_len(c):
            cid = wid + NW * c
            return cid, t_tab[cid // CPG]

        def chunk_tiles(c):
            ntiles = nt_tab[(wid + NW * c) // CPG]
            return ntiles, ntiles >= NT - 1

        def start_in(c, b):
            cid = wid + NW * c
            ntiles, full = chunk_tiles(c)
            row = cid * R_CHUNK

            @pl.when(full)
            def _full():
                pltpu.async_copy(inp_hbm.at[pl.ds(row, R_CHUNK)],
                                 ins[b], isems[b])

            @pl.when(~full)
            def _prefix():
                def t_body(t, carry):
                    pltpu.async_copy(
                        inp_hbm.at[pl.ds(row, R_CHUNK),
                                   pl.ds(t * W_TILE, W_TILE)],
                        ins[b].at[:, pl.ds(t * W_TILE, W_TILE)], isems[b])
                    return carry
                lax.fori_loop(0, ntiles, t_body, 0)

        def wait_in(c, b):
            ntiles, full = chunk_tiles(c)

            @pl.when(full)
            def _full():
                pltpu.make_async_copy(
                    inp_hbm.at[pl.ds(0, R_CHUNK)], ins[b], isems[b]).wait()

            @pl.when(~full)
            def _prefix():
                def t_body(t, carry):
                    pltpu.make_async_copy(
                        inp_hbm.at[pl.ds(0, R_CHUNK), pl.ds(0, W_TILE)],
                        ins[b].at[:, pl.ds(0, W_TILE)], isems[b]).wait()
                    return carry
                lax.fori_loop(0, ntiles, t_body, 0)

        def start_out(c, b):
            cid = wid + NW * c
            pltpu.async_copy(
                obs[b], out_hbm.at[pl.ds(cid * R_CHUNK, R_CHUNK)], osems[b])

        def wait_out(b):
            pltpu.make_async_copy(
                obs[b], out_hbm.at[pl.ds(0, R_CHUNK)], osems[b]).wait()

        def compute(c, b):
            _, T = chunk_len(c)
            cols = [_round_idx((T - 1) * (lane + v * LANES))
                    for v in range(N_SEG // LANES)]
            buf, ob = ins[b], obs[b]
            for r in range(R_CHUNK):
                rsp = jnp.full((LANES,), r, jnp.int32)
                vecs = [plsc.load_gather(buf, [rsp, cols[v]])
                        for v in range(N_SEG // LANES)]
                for v, vec in enumerate(vecs):
                    ob[r, pl.ds(v * LANES, LANES)] = vec

        start_in(0, 0)
        start_in(1, 1)

        def body(i, carry):
            c = 3 * i
            start_in(c + 2, 2)

            wait_in(c, 0)
            @pl.when(i > 0)
            def _w0():
                wait_out(0)
            compute(c, 0)
            start_out(c, 0)
            start_in(c + 3, 0)

            wait_in(c + 1, 1)
            @pl.when(i > 0)
            def _w1():
                wait_out(1)
            compute(c + 1, 1)
            start_out(c + 1, 1)
            start_in(c + 4, 1)

            wait_in(c + 2, 2)
            @pl.when(i > 0)
            def _w2():
                wait_out(2)
            compute(c + 2, 2)
            start_out(c + 2, 2)
            return carry

        NI = N_CHUNK // 3
        lax.fori_loop(0, NI, body, 0)

        wait_in(N_CHUNK - 2, 0)
        wait_out(0)
        compute(N_CHUNK - 2, 0)
        start_out(N_CHUNK - 2, 0)

        wait_in(N_CHUNK - 1, 1)
        wait_out(1)
        compute(N_CHUNK - 1, 1)
        start_out(N_CHUNK - 1, 1)

        wait_out(2)
        wait_out(0)
        wait_out(1)

    return k


_sc_kernel = _make_sc_kernel()


def kernel(inp, length, n_batchs):
    del n_batchs
    return _sc_kernel(inp, length.astype(jnp.int32))


# speedup vs baseline: 1.0605x; 1.0605x over previous
import functools

import jax
import jax.numpy as jnp
from jax import lax
from jax.experimental import pallas as pl
from jax.experimental.pallas import tpu as pltpu
from jax.experimental.pallas import tpu_sc as plsc

N_SEG = 64
LANES = 16
NC, NS = 2, 16
NW = NC * NS
ROWS = 16384
COLS = 2048
R_CHUNK = 16
N_CHUNK = ROWS // R_CHUNK // NW
CPG = 1024 // R_CHUNK
W_TILE = 256
W_SHIFT = W_TILE.bit_length() - 1
NT = COLS // W_TILE


def _round_idx(num):
    q = num >> 6
    rem = num & 63
    tie_up = (rem == 32) & ((q & 1) == 0)
    inc = jnp.where((rem > 32) | tie_up, 1, 0)
    return q + inc


def _make_sc_kernel():
    mesh = plsc.VectorSubcoreMesh(core_axis_name="c", subcore_axis_name="s")

    @functools.partial(
        pl.kernel,
        mesh=mesh,
        compiler_params=pltpu.CompilerParams(needs_layout_passes=False),
        out_type=jax.ShapeDtypeStruct((ROWS, N_SEG), jnp.float32),
        scratch_types=[
            pltpu.VMEM((LANES,), jnp.int32),
            pltpu.VMEM((R_CHUNK, COLS), jnp.float32),
            pltpu.VMEM((R_CHUNK, COLS), jnp.float32),
            pltpu.VMEM((R_CHUNK, COLS), jnp.float32),
            pltpu.VMEM((R_CHUNK, N_SEG), jnp.float32),
            pltpu.VMEM((R_CHUNK, N_SEG), jnp.float32),
            pltpu.VMEM((R_CHUNK, N_SEG), jnp.float32),
            pltpu.SMEM((LANES,), jnp.int32),
            pltpu.SMEM((LANES,), jnp.int32),
            pltpu.SemaphoreType.DMA,
            pltpu.SemaphoreType.DMA,
            pltpu.SemaphoreType.DMA,
            pltpu.SemaphoreType.DMA,
            pltpu.SemaphoreType.DMA,
            pltpu.SemaphoreType.DMA,
        ],
    )
    def k(inp_hbm, len_hbm, out_hbm, len_v,
          in0, in1, in2, ob0, ob1, ob2, t_tab, nt_tab,
          is0, is1, is2, os0, os1, os2):
        wid = lax.axis_index("s") * NC + lax.axis_index("c")

        pltpu.sync_copy(len_hbm, len_v)
        lane = lax.iota(jnp.int32, LANES)

        ins = [in0, in1, in2]
        obs = [ob0, ob1, ob2]
        isems = [is0, is1, is2]
        osems = [os0, os1, os2]

        def fill_tab(g, carry):
            T0 = jnp.max(jnp.where(lane == g, len_v[...], 0))
            T = jnp.where(T0 < 2 * N_SEG, (2 * N_SEG // T0 + 1) * T0, T0)
            idx_max = _round_idx((T - 1) * (N_SEG - 1))
            t_tab[g] = T
            nt_tab[g] = (idx_max >> W_SHIFT) + 1
            return carry
        lax.fori_loop(0, LANES, fill_tab, 0)

        def chunk_len(c):
            cid = wid + NW * c
            return cid, t_tab[cid // CPG]

        def chunk_tiles(c):
            ntiles = nt_tab[(wid + NW * c) // CPG]
            return ntiles, ntiles >= NT

        def start_in(c, b):
            cid = wid + NW * c
            ntiles, full = chunk_tiles(c)
            row = cid * R_CHUNK

            @pl.when(full)
            def _full():
                pltpu.async_copy(inp_hbm.at[pl.ds(row, R_CHUNK)],
                                 ins[b], isems[b])

            @pl.when(~full)
            def _prefix():
                def t_body(t, carry):
                    pltpu.async_copy(
                        inp_hbm.at[pl.ds(row, R_CHUNK),
                                   pl.ds(t * W_TILE, W_TILE)],
                        ins[b].at[:, pl.ds(t * W_TILE, W_TILE)], isems[b])
                    return carry
                lax.fori_loop(0, ntiles, t_body, 0)

        def wait_in(c, b):
            ntiles, full = chunk_tiles(c)

            @pl.when(full)
            def _full():
                pltpu.make_async_copy(
                    inp_hbm.at[pl.ds(0, R_CHUNK)], ins[b], isems[b]).wait()

            @pl.when(~full)
            def _prefix():
                def t_body(t, carry):
                    pltpu.make_async_copy(
                        inp_hbm.at[pl.ds(0, R_CHUNK), pl.ds(0, W_TILE)],
                        ins[b].at[:, pl.ds(0, W_TILE)], isems[b]).wait()
                    return carry
                lax.fori_loop(0, ntiles, t_body, 0)

        def start_out(c, b):
            cid = wid + NW * c
            pltpu.async_copy(
                obs[b], out_hbm.at[pl.ds(cid * R_CHUNK, R_CHUNK)], osems[b])

        def wait_out(b):
            pltpu.make_async_copy(
                obs[b], out_hbm.at[pl.ds(0, R_CHUNK)], osems[b]).wait()

        def compute(c, b):
            _, T = chunk_len(c)
            cols = [_round_idx((T - 1) * (lane + v * LANES))
                    for v in range(N_SEG // LANES)]
            buf, ob = ins[b], obs[b]
            for r in range(R_CHUNK):
                rsp = jnp.full((LANES,), r, jnp.int32)
                vecs = [plsc.load_gather(buf, [rsp, cols[v]])
                        for v in range(N_SEG // LANES)]
                for v, vec in enumerate(vecs):
                    ob[r, pl.ds(v * LANES, LANES)] = vec

        start_in(0, 0)
        start_in(1, 1)

        def body(i, carry):
            c = 3 * i
            start_in(c + 2, 2)

            wait_in(c, 0)
            @pl.when(i > 0)
            def _w0():
                wait_out(0)
            compute(c, 0)
            start_out(c, 0)
            start_in(c + 3, 0)

            wait_in(c + 1, 1)
            @pl.when(i > 0)
            def _w1():
                wait_out(1)
            compute(c + 1, 1)
            start_out(c + 1, 1)
            start_in(c + 4, 1)

            wait_in(c + 2, 2)
            @pl.when(i > 0)
            def _w2():
                wait_out(2)
            compute(c + 2, 2)
            start_out(c + 2, 2)
            return carry

        NI = N_CHUNK // 3
        lax.fori_loop(0, NI, body, 0)

        wait_in(N_CHUNK - 2, 0)
        wait_out(0)
        compute(N_CHUNK - 2, 0)
        start_out(N_CHUNK - 2, 0)

        wait_in(N_CHUNK - 1, 1)
        wait_out(1)
        compute(N_CHUNK - 1, 1)
        start_out(N_CHUNK - 1, 1)

        wait_out(2)
        wait_out(0)
        wait_out(1)

    return k


_sc_kernel = _make_sc_kernel()


def kernel(inp, length, n_batchs):
    del n_batchs
    return _sc_kernel(inp, length.astype(jnp.int32))


# speedup vs baseline: 1.0910x; 1.0288x over previous
import functools

import jax
import jax.numpy as jnp
from jax import lax
from jax.experimental import pallas as pl
from jax.experimental.pallas import tpu as pltpu
from jax.experimental.pallas import tpu_sc as plsc

N_SEG = 64
LANES = 16
NC, NS = 2, 16
NW = NC * NS
ROWS = 16384
COLS = 2048
R_CHUNK = 16
N_CHUNK = ROWS // R_CHUNK // NW
CPG = 1024 // R_CHUNK
W_TILE = 128
W_SHIFT = W_TILE.bit_length() - 1
NT = COLS // W_TILE


def _round_idx(num):
    q = num >> 6
    rem = num & 63
    tie_up = (rem == 32) & ((q & 1) == 0)
    inc = jnp.where((rem > 32) | tie_up, 1, 0)
    return q + inc


def _make_sc_kernel():
    mesh = plsc.VectorSubcoreMesh(core_axis_name="c", subcore_axis_name="s")

    @functools.partial(
        pl.kernel,
        mesh=mesh,
        compiler_params=pltpu.CompilerParams(needs_layout_passes=False),
        out_type=jax.ShapeDtypeStruct((ROWS, N_SEG), jnp.float32),
        scratch_types=[
            pltpu.VMEM((LANES,), jnp.int32),
            pltpu.VMEM((R_CHUNK, COLS), jnp.float32),
            pltpu.VMEM((R_CHUNK, COLS), jnp.float32),
            pltpu.VMEM((R_CHUNK, COLS), jnp.float32),
            pltpu.VMEM((R_CHUNK, N_SEG), jnp.float32),
            pltpu.VMEM((R_CHUNK, N_SEG), jnp.float32),
            pltpu.VMEM((R_CHUNK, N_SEG), jnp.float32),
            pltpu.SMEM((LANES,), jnp.int32),
            pltpu.SMEM((LANES,), jnp.int32),
            pltpu.SemaphoreType.DMA,
            pltpu.SemaphoreType.DMA,
            pltpu.SemaphoreType.DMA,
            pltpu.SemaphoreType.DMA,
            pltpu.SemaphoreType.DMA,
            pltpu.SemaphoreType.DMA,
        ],
    )
    def k(inp_hbm, len_hbm, out_hbm, len_v,
          in0, in1, in2, ob0, ob1, ob2, t_tab, nt_tab,
          is0, is1, is2, os0, os1, os2):
        wid = lax.axis_index("s") * NC + lax.axis_index("c")

        pltpu.sync_copy(len_hbm, len_v)
        lane = lax.iota(jnp.int32, LANES)

        ins = [in0, in1, in2]
        obs = [ob0, ob1, ob2]
        isems = [is0, is1, is2]
        osems = [os0, os1, os2]

        def fill_tab(g, carry):
            T0 = jnp.max(jnp.where(lane == g, len_v[...], 0))
            T = jnp.where(T0 < 2 * N_SEG, (2 * N_SEG // T0 + 1) * T0, T0)
            idx_max = _round_idx((T - 1) * (N_SEG - 1))
            t_tab[g] = T
            nt_tab[g] = (idx_max >> W_SHIFT) + 1
            return carry
        lax.fori_loop(0, LANES, fill_tab, 0)

        def chunk_len(c):
            cid = wid + NW * c
            return cid, t_tab[cid // CPG]

        def chunk_tiles(c):
            ntiles = nt_tab[(wid + NW * c) // CPG]
            return ntiles, ntiles >= NT

        def start_in(c, b):
            cid = wid + NW * c
            ntiles, full = chunk_tiles(c)
            row = cid * R_CHUNK

            @pl.when(full)
            def _full():
                pltpu.async_copy(inp_hbm.at[pl.ds(row, R_CHUNK)],
                                 ins[b], isems[b])

            @pl.when(~full)
            def _prefix():
                def t_body(t, carry):
                    pltpu.async_copy(
                        inp_hbm.at[pl.ds(row, R_CHUNK),
                                   pl.ds(t * W_TILE, W_TILE)],
                        ins[b].at[:, pl.ds(t * W_TILE, W_TILE)], isems[b])
                    return carry
                lax.fori_loop(0, ntiles, t_body, 0)

        def wait_in(c, b):
            ntiles, full = chunk_tiles(c)

            @pl.when(full)
            def _full():
                pltpu.make_async_copy(
                    inp_hbm.at[pl.ds(0, R_CHUNK)], ins[b], isems[b]).wait()

            @pl.when(~full)
            def _prefix():
                def t_body(t, carry):
                    pltpu.make_async_copy(
                        inp_hbm.at[pl.ds(0, R_CHUNK), pl.ds(0, W_TILE)],
                        ins[b].at[:, pl.ds(0, W_TILE)], isems[b]).wait()
                    return carry
                lax.fori_loop(0, ntiles, t_body, 0)

        def start_out(c, b):
            cid = wid + NW * c
            pltpu.async_copy(
                obs[b], out_hbm.at[pl.ds(cid * R_CHUNK, R_CHUNK)], osems[b])

        def wait_out(b):
            pltpu.make_async_copy(
                obs[b], out_hbm.at[pl.ds(0, R_CHUNK)], osems[b]).wait()

        def compute(c, b):
            _, T = chunk_len(c)
            cols = [_round_idx((T - 1) * (lane + v * LANES))
                    for v in range(N_SEG // LANES)]
            buf, ob = ins[b], obs[b]
            for r in range(R_CHUNK):
                rsp = jnp.full((LANES,), r, jnp.int32)
                vecs = [plsc.load_gather(buf, [rsp, cols[v]])
                        for v in range(N_SEG // LANES)]
                for v, vec in enumerate(vecs):
                    ob[r, pl.ds(v * LANES, LANES)] = vec

        start_in(0, 0)
        start_in(1, 1)

        def body(i, carry):
            c = 3 * i
            start_in(c + 2, 2)

            wait_in(c, 0)
            @pl.when(i > 0)
            def _w0():
                wait_out(0)
            compute(c, 0)
            start_out(c, 0)
            start_in(c + 3, 0)

            wait_in(c + 1, 1)
            @pl.when(i > 0)
            def _w1():
                wait_out(1)
            compute(c + 1, 1)
            start_out(c + 1, 1)
            start_in(c + 4, 1)

            wait_in(c + 2, 2)
            @pl.when(i > 0)
            def _w2():
                wait_out(2)
            compute(c + 2, 2)
            start_out(c + 2, 2)
            return carry

        NI = N_CHUNK // 3
        lax.fori_loop(0, NI, body, 0)

        wait_in(N_CHUNK - 2, 0)
        wait_out(0)
        compute(N_CHUNK - 2, 0)
        start_out(N_CHUNK - 2, 0)

        wait_in(N_CHUNK - 1, 1)
        wait_out(1)
        compute(N_CHUNK - 1, 1)
        start_out(N_CHUNK - 1, 1)

        wait_out(2)
        wait_out(0)
        wait_out(1)

    return k


_sc_kernel = _make_sc_kernel()


def kernel(inp, length, n_batchs):
    del n_batchs
    return _sc_kernel(inp, length.astype(jnp.int32))
